# baseline (speedup 1.0000x reference)
import functools

import jax
import jax.numpy as jnp
from jax import lax
from jax.experimental import pallas as pl
from jax.experimental.pallas import tpu as pltpu
from jax.experimental.pallas import tpu_sc as plsc

N = 10000
D = 128
E = 320000
NC = 2
NS = 16
NW = NC * NS
EPT = E // NW
C = 128
NCHUNK = 79
EPT_P = NCHUNK * C
E_P = NW * EPT_P
N_PAD = 10240
STRIPE = N_PAD // NS
EXP_NEG_GAMMA = 0.36787944117144233

_mesh = plsc.VectorSubcoreMesh(
    core_axis_name="c", subcore_axis_name="s", num_cores=NC, num_subcores=NS
)
_sc_params = pltpu.CompilerParams(needs_layout_passes=False)


@functools.partial(
    pl.kernel,
    out_type=(
        jax.ShapeDtypeStruct((E_P,), jnp.float32),
        jax.ShapeDtypeStruct((NC, N_PAD), jnp.float32),
    ),
    mesh=_mesh,
    scratch_types=[
        pltpu.VMEM((N_PAD,), jnp.float32),
        pltpu.VMEM((N_PAD,), jnp.int32),
        pltpu.VMEM((2, C), jnp.int32),
        pltpu.VMEM((C,), jnp.int32),
        pltpu.VMEM((C,), jnp.int32),
        pltpu.VMEM((C, D), jnp.float32),
        pltpu.VMEM((C, D), jnp.float32),
        pltpu.VMEM((C, D), jnp.float32),
        pltpu.VMEM((C, D), jnp.float32),
        pltpu.VMEM((256,), jnp.float32),
        pltpu.VMEM((C,), jnp.float32),
        pltpu.VMEM((C,), jnp.float32),
        pltpu.VMEM((STRIPE,), jnp.float32),
        pltpu.VMEM_SHARED((N_PAD,), jnp.float32),
        pltpu.SemaphoreType.DMA,
        pltpu.SemaphoreType.DMA,
    ],
    compiler_params=_sc_params,
)
def _edge_weight_kernel(x_hbm, src_hbm, dst_hbm, nrm_hbm, sa_hbm,
                        ew_hbm, deg_hbm,
                        nrm_t, sa_t, sidx2, didx0, didx1,
                        xs0, xs1, xd0, xd1, pbuf, ewb0, ewb1,
                        zstripe, deg_sh, gsem0, gsem1):
    cid = lax.axis_index("c")
    sid = lax.axis_index("s")
    wid = sid * NC + cid
    didx = (didx0, didx1)
    xs = (xs0, xs1)
    xd = (xd0, xd1)
    ewb = (ewb0, ewb1)
    gsem = (gsem0, gsem1)

    pltpu.sync_copy(nrm_hbm, nrm_t)
    pltpu.sync_copy(sa_hbm, sa_t)

    zero16 = jnp.zeros((16,), jnp.float32)

    def _zero(i, carry):
        zstripe[pl.ds(i * 16, 16)] = zero16
        return carry

    lax.fori_loop(0, STRIPE // 16, _zero, 0)
    pltpu.sync_copy(zstripe, deg_sh.at[pl.ds(sid * STRIPE, STRIPE)])
    plsc.subcore_barrier()

    base = wid * EPT_P
    lanes = lax.broadcasted_iota(jnp.int32, (16,), 0)
    lanes16 = lanes * 16

    def _load_idx(slot, c):
        eb = base + c * C
        pltpu.sync_copy(src_hbm.at[pl.ds(eb, C)], sidx2.at[slot])
        pltpu.sync_copy(dst_hbm.at[pl.ds(eb, C)], didx[slot])

    def _issue_gathers(slot):
        h1 = pltpu.async_copy(x_hbm.at[sidx2.at[slot]], xs[slot], gsem[slot])
        h2 = pltpu.async_copy(x_hbm.at[didx[slot]], xd[slot], gsem[slot])
        return h1, h2

    def _compute(slot, c):
        eb = base + c * C
        xsb = xs[slot]
        xdb = xd[slot]
        for g in range(C // 16):
            sl = pl.ds(g * 16, 16)
            sv = sidx2[slot, sl]
            dv = didx[slot][sl]
            ns = plsc.load_gather(nrm_t, [sv])
            nd = plsc.load_gather(nrm_t, [dv])
            sas = plsc.load_gather(sa_t, [sv])
            sad = plsc.load_gather(sa_t, [dv])

            def _dj(j, carry):
                e = g * 16 + j
                acc = xsb[e, pl.ds(0, 16)] * xdb[e, pl.ds(0, 16)]
                for k in range(1, 8):
                    acc = acc + xsb[e, pl.ds(k * 16, 16)] \
                        * xdb[e, pl.ds(k * 16, 16)]
                plsc.store_scatter(pbuf, [lanes16 + j], acc)
                return carry

            lax.fori_loop(0, 16, _dj, 0, unroll=4)
            dotv = pbuf[pl.ds(0, 16)]
            for l in range(1, 16):
                dotv = dotv + pbuf[pl.ds(l * 16, 16)]
            sim = dotv / jnp.maximum(ns * nd, 1e-8)
            ewv = jnp.where(sas != sad, sim * EXP_NEG_GAMMA, sim)
            ewb[slot][sl] = jnp.maximum(ewv, 1e-4)
        pltpu.sync_copy(ewb[slot], ew_hbm.at[pl.ds(eb, C)])
        pltpu.sync_copy(ewb[slot], deg_sh.at[didx[slot]], add=True)

    _load_idx(0, 0)
    h1, h2 = _issue_gathers(0)
    h1.wait()
    h2.wait()

    def _pair(i, carry):
        c0 = i * 2
        _load_idx(1, c0 + 1)
        ha, hb = _issue_gathers(1)
        _compute(0, c0)
        ha.wait()
        hb.wait()
        _load_idx(0, c0 + 2)
        hc, hd = _issue_gathers(0)
        _compute(1, c0 + 1)
        hc.wait()
        hd.wait()
        return carry

    lax.fori_loop(0, (NCHUNK - 1) // 2, _pair, 0)
    _compute(0, NCHUNK - 1)

    plsc.subcore_barrier()
    pltpu.sync_copy(deg_sh.at[pl.ds(sid * STRIPE, STRIPE)],
                    deg_hbm.at[cid, pl.ds(sid * STRIPE, STRIPE)])


@functools.partial(
    pl.kernel,
    out_type=jax.ShapeDtypeStruct((NC, N_PAD, D), jnp.float32),
    mesh=_mesh,
    scratch_types=[
        pltpu.VMEM((N_PAD,), jnp.float32),
        pltpu.VMEM((2, C), jnp.int32),
        pltpu.VMEM((C,), jnp.int32),
        pltpu.VMEM((C,), jnp.int32),
        pltpu.VMEM((2, C), jnp.float32),
        pltpu.VMEM((C,), jnp.float32),
        pltpu.VMEM((C, D), jnp.float32),
        pltpu.VMEM((C, D), jnp.float32),
        pltpu.VMEM_SHARED((N_PAD, D), jnp.float32),
        pltpu.SemaphoreType.DMA,
        pltpu.SemaphoreType.DMA,
    ],
    compiler_params=_sc_params,
)
def _aggregate_kernel(feat_hbm, src_hbm, dst_hbm, ew_in_hbm, dinv_hbm,
                      zeros_hbm, out_hbm,
                      dinv_t, sidx2, didx0, didx1, ewc2, normc,
                      ra0, ra1, acc_sh,
                      gsem0, gsem1):
    cid = lax.axis_index("c")
    sid = lax.axis_index("s")
    wid = sid * NC + cid
    didx = (didx0, didx1)
    ra = (ra0, ra1)
    gsem = (gsem0, gsem1)

    pltpu.sync_copy(dinv_hbm, dinv_t)
    row0 = sid * STRIPE
    pltpu.sync_copy(zeros_hbm, acc_sh.at[pl.ds(row0, STRIPE)])
    plsc.subcore_barrier()

    base = wid * EPT_P

    def _load_idx(slot, c):
        eb = base + c * C
        pltpu.sync_copy(src_hbm.at[pl.ds(eb, C)], sidx2.at[slot])
        pltpu.sync_copy(dst_hbm.at[pl.ds(eb, C)], didx[slot])
        pltpu.sync_copy(ew_in_hbm.at[pl.ds(eb, C)], ewc2.at[slot])

    def _issue_gather(slot):
        return pltpu.async_copy(feat_hbm.at[sidx2.at[slot]], ra[slot],
                                gsem[slot])

    def _compute(slot):
        rab = ra[slot]
        for g in range(C // 16):
            sl = pl.ds(g * 16, 16)
            dis = plsc.load_gather(dinv_t, [sidx2[slot, sl]])
            normc[sl] = dis * ewc2[slot, sl]

        def _scale(e, carry):
            s16 = plsc.load_gather(normc, [jnp.full((16,), e, jnp.int32)])
            for k in range(D // 16):
                sl2 = pl.ds(k * 16, 16)
                rab[e, sl2] = rab[e, sl2] * s16
            return carry

        lax.fori_loop(0, C, _scale, 0, unroll=4)
        pltpu.sync_copy(rab, acc_sh.at[didx[slot]], add=True)

    _load_idx(0, 0)
    _issue_gather(0).wait()

    def _pair(i, carry):
        c0 = i * 2
        _load_idx(1, c0 + 1)
        ha = _issue_gather(1)
        _compute(0)
        ha.wait()
        _load_idx(0, c0 + 2)
        hb = _issue_gather(0)
        _compute(1)
        hb.wait()
        return carry

    lax.fori_loop(0, (NCHUNK - 1) // 2, _pair, 0)
    _compute(0)

    plsc.subcore_barrier()
    for k5 in range(STRIPE // 64):
        sl = pl.ds(row0 + k5 * 64, 64)
        pltpu.sync_copy(acc_sh.at[sl], out_hbm.at[cid, sl])


def _rownorm_body(x_ref, nrm_ref):
    x = x_ref[...]
    nrm_ref[...] = jnp.sqrt(jnp.sum(x * x, axis=1, keepdims=True))


def _row_norms(x_pad):
    rb = 1024
    return pl.pallas_call(
        _rownorm_body,
        grid=(N_PAD // rb,),
        in_specs=[pl.BlockSpec((rb, D), lambda i: (i, 0))],
        out_specs=pl.BlockSpec((rb, 1), lambda i: (i, 0)),
        out_shape=jax.ShapeDtypeStruct((N_PAD, 1), jnp.float32),
    )(x_pad)


def _dinv_body(d0_ref, d1_ref, dinv_ref, dinv2_ref):
    deg = d0_ref[...] + d1_ref[...] + 1.0
    di = jnp.where(deg > 0, lax.rsqrt(deg), 0.0)
    dinv_ref[...] = di
    dinv2_ref[...] = di * di


def _compute_dinv(deg_parts):
    d0 = deg_parts[0].reshape(80, 128)
    d1 = deg_parts[1].reshape(80, 128)
    dinv, dinv2 = pl.pallas_call(
        _dinv_body,
        in_specs=[pl.BlockSpec((80, 128), lambda: (0, 0))] * 2,
        out_specs=[pl.BlockSpec((80, 128), lambda: (0, 0))] * 2,
        out_shape=(jax.ShapeDtypeStruct((80, 128), jnp.float32),) * 2,
    )(d0, d1)
    return dinv.reshape(N_PAD), dinv.reshape(N_PAD, 1), dinv2.reshape(N_PAD, 1)


def _combine_body(p0_ref, p1_ref, f_ref, d1_ref, d2_ref, w_ref, b_ref,
                  out_ref):
    z = d1_ref[...] * (p0_ref[...] + p1_ref[...]) + d2_ref[...] * f_ref[...]
    o = jnp.dot(z, w_ref[...], preferred_element_type=jnp.float32) + b_ref[...]
    out_ref[...] = jnp.maximum(o, 0.0)


def _combine_matmul_relu(parts, feat_pad, dinv1, dinv2, W, b):
    rb = 1024
    return pl.pallas_call(
        _combine_body,
        grid=(N_PAD // rb,),
        in_specs=[
            pl.BlockSpec((rb, D), lambda i: (i, 0)),
            pl.BlockSpec((rb, D), lambda i: (i, 0)),
            pl.BlockSpec((rb, D), lambda i: (i, 0)),
            pl.BlockSpec((rb, 1), lambda i: (i, 0)),
            pl.BlockSpec((rb, 1), lambda i: (i, 0)),
            pl.BlockSpec((D, D), lambda i: (0, 0)),
            pl.BlockSpec((1, D), lambda i: (0, 0)),
        ],
        out_specs=pl.BlockSpec((rb, D), lambda i: (i, 0)),
        out_shape=jax.ShapeDtypeStruct((N_PAD, D), jnp.float32),
    )(parts[0], parts[1], feat_pad, dinv1, dinv2, W, b.reshape(1, D))


def _final_body(p0_ref, p1_ref, f_ref, d1_ref, d2_ref, w_ref, b_ref,
                wc_ref, bc_ref, h_ref, y_ref):
    z = d1_ref[...] * (p0_ref[...] + p1_ref[...]) + d2_ref[...] * f_ref[...]
    o = jnp.dot(z, w_ref[...], preferred_element_type=jnp.float32) + b_ref[...]
    h = jnp.maximum(o, 0.0)
    h_ref[...] = h
    y_ref[...] = jnp.dot(h, wc_ref[...], preferred_element_type=jnp.float32) \
        + bc_ref[...]


def _final_layer(parts, feat_pad, dinv1, dinv2, W, b, Wc, bc):
    rb = 1024
    return pl.pallas_call(
        _final_body,
        grid=(N_PAD // rb,),
        in_specs=[
            pl.BlockSpec((rb, D), lambda i: (i, 0)),
            pl.BlockSpec((rb, D), lambda i: (i, 0)),
            pl.BlockSpec((rb, D), lambda i: (i, 0)),
            pl.BlockSpec((rb, 1), lambda i: (i, 0)),
            pl.BlockSpec((rb, 1), lambda i: (i, 0)),
            pl.BlockSpec((D, D), lambda i: (0, 0)),
            pl.BlockSpec((1, D), lambda i: (0, 0)),
            pl.BlockSpec((D, 1), lambda i: (0, 0)),
            pl.BlockSpec((1, 1), lambda i: (0, 0)),
        ],
        out_specs=[
            pl.BlockSpec((rb, D), lambda i: (i, 0)),
            pl.BlockSpec((rb, 1), lambda i: (i, 0)),
        ],
        out_shape=(
            jax.ShapeDtypeStruct((N_PAD, D), jnp.float32),
            jax.ShapeDtypeStruct((N_PAD, 1), jnp.float32),
        ),
    )(parts[0], parts[1], feat_pad, dinv1, dinv2, W, b.reshape(1, D),
      Wc, bc.reshape(1, 1))


def kernel(x, edge_index, sensitive_attr, W1, b1, W2, b2, Wc, bc):
    edge_index = edge_index.astype(jnp.int32)
    ei = edge_index.reshape(2, NW, EPT)
    ei = jnp.pad(ei, ((0, 0), (0, 0), (0, EPT_P - EPT)),
                 constant_values=N_PAD - 1)
    src = ei[0].reshape(E_P)
    dst = ei[1].reshape(E_P)
    sa = sensitive_attr.astype(jnp.int32)
    sa_pad = jnp.pad(sa, (0, N_PAD - N))
    x_pad = jnp.pad(x, ((0, N_PAD - N), (0, 0)))

    nrm = _row_norms(x_pad).reshape(N_PAD)
    ew, deg_parts = _edge_weight_kernel(x_pad, src, dst, nrm, sa_pad)
    dinv, dinv1, dinv2 = _compute_dinv(deg_parts)

    zeros_acc = jnp.zeros((STRIPE, D), jnp.float32)
    agg1 = _aggregate_kernel(x_pad, src, dst, ew, dinv, zeros_acc)
    h1 = _combine_matmul_relu(agg1, x_pad, dinv1, dinv2, W1, b1)

    agg2 = _aggregate_kernel(h1, src, dst, ew, dinv, zeros_acc)
    h2, y = _final_layer(agg2, h1, dinv1, dinv2, W2, b2, Wc, bc)

    return (y[:N], h2[:N])


# speedup vs baseline: 1.4678x; 1.4678x over previous
import functools

import jax
import jax.numpy as jnp
from jax import lax
from jax.experimental import pallas as pl
from jax.experimental.pallas import tpu as pltpu
from jax.experimental.pallas import tpu_sc as plsc

N = 10000
D = 128
E = 320000
NC = 2
NS = 16
NW = NC * NS
EPT = E // NW
C = 80
NCHUNK = EPT // C
N_PAD = 10240
STRIPE = N_PAD // NS
EXP_NEG_GAMMA = 0.36787944117144233

_mesh = plsc.VectorSubcoreMesh(
    core_axis_name="c", subcore_axis_name="s", num_cores=NC, num_subcores=NS
)
_sc_params = pltpu.CompilerParams(needs_layout_passes=False)


@functools.partial(
    pl.kernel,
    out_type=(
        jax.ShapeDtypeStruct((E,), jnp.float32),
        jax.ShapeDtypeStruct((NC, N_PAD), jnp.float32),
    ),
    mesh=_mesh,
    scratch_types=[
        pltpu.VMEM((N_PAD,), jnp.float32),
        pltpu.VMEM((N_PAD,), jnp.int32),
        pltpu.VMEM((2, 2, C), jnp.int32),
        pltpu.VMEM((C, D), jnp.float32),
        pltpu.VMEM((C, D), jnp.float32),
        pltpu.VMEM((C, D), jnp.float32),
        pltpu.VMEM((C, D), jnp.float32),
        pltpu.VMEM((256,), jnp.float32),
        pltpu.VMEM((C,), jnp.float32),
        pltpu.VMEM((C,), jnp.float32),
        pltpu.VMEM((STRIPE,), jnp.float32),
        pltpu.VMEM_SHARED((N_PAD,), jnp.float32),
        pltpu.SemaphoreType.DMA,
        pltpu.SemaphoreType.DMA,
    ],
    compiler_params=_sc_params,
)
def _edge_weight_kernel(x_hbm, sd_hbm, nrm_hbm, sa_hbm,
                        ew_hbm, deg_hbm,
                        nrm_t, sa_t, sd,
                        xs0, xs1, xd0, xd1, pbuf, ewb0, ewb1,
                        zstripe, deg_sh, gsem0, gsem1):
    cid = lax.axis_index("c")
    sid = lax.axis_index("s")
    wid = sid * NC + cid
    xs = (xs0, xs1)
    xd = (xd0, xd1)
    ewb = (ewb0, ewb1)
    gsem = (gsem0, gsem1)

    pltpu.sync_copy(nrm_hbm, nrm_t)
    pltpu.sync_copy(sa_hbm, sa_t)

    zero16 = jnp.zeros((16,), jnp.float32)

    def _zero(i, carry):
        zstripe[pl.ds(i * 16, 16)] = zero16
        return carry

    lax.fori_loop(0, STRIPE // 16, _zero, 0)
    pltpu.sync_copy(zstripe, deg_sh.at[pl.ds(sid * STRIPE, STRIPE)])
    plsc.subcore_barrier()

    base = wid * EPT
    tbase = wid * NCHUNK
    lanes = lax.broadcasted_iota(jnp.int32, (16,), 0)
    lanes16 = lanes * 16

    def _load_idx(slot, c):
        pltpu.sync_copy(sd_hbm.at[tbase + c], sd.at[slot])

    def _issue_gathers(slot):
        h1 = pltpu.async_copy(x_hbm.at[sd.at[slot, 0]], xs[slot], gsem[slot])
        h2 = pltpu.async_copy(x_hbm.at[sd.at[slot, 1]], xd[slot], gsem[slot])
        return h1, h2

    def _compute(slot, c):
        eb = base + c * C
        xsb = xs[slot]
        xdb = xd[slot]
        for g in range(C // 16):
            sl = pl.ds(g * 16, 16)
            sv = sd[slot, 0, sl]
            dv = sd[slot, 1, sl]
            ns = plsc.load_gather(nrm_t, [sv])
            nd = plsc.load_gather(nrm_t, [dv])
            sas = plsc.load_gather(sa_t, [sv])
            sad = plsc.load_gather(sa_t, [dv])

            def _dj(j, carry):
                e = g * 16 + j
                acc = xsb[e, pl.ds(0, 16)] * xdb[e, pl.ds(0, 16)]
                for k in range(1, 8):
                    acc = acc + xsb[e, pl.ds(k * 16, 16)] \
                        * xdb[e, pl.ds(k * 16, 16)]
                plsc.store_scatter(pbuf, [lanes16 + j], acc)
                return carry

            lax.fori_loop(0, 16, _dj, 0, unroll=4)
            dotv = pbuf[pl.ds(0, 16)]
            for l in range(1, 16):
                dotv = dotv + pbuf[pl.ds(l * 16, 16)]
            sim = dotv / jnp.maximum(ns * nd, 1e-8)
            ewv = jnp.where(sas != sad, sim * EXP_NEG_GAMMA, sim)
            ewb[slot][sl] = jnp.maximum(ewv, 1e-4)
        pltpu.sync_copy(ewb[slot], ew_hbm.at[pl.ds(eb, C)])
        pltpu.sync_copy(ewb[slot], deg_sh.at[sd.at[slot, 1]], add=True)

    _load_idx(0, 0)
    h1, h2 = _issue_gathers(0)
    h1.wait()
    h2.wait()

    def _pair(i, carry):
        c0 = i * 2
        _load_idx(1, c0 + 1)
        ha, hb = _issue_gathers(1)
        _compute(0, c0)
        ha.wait()
        hb.wait()
        _load_idx(0, c0 + 2)
        hc, hd = _issue_gathers(0)
        _compute(1, c0 + 1)
        hc.wait()
        hd.wait()
        return carry

    lax.fori_loop(0, (NCHUNK - 1) // 2, _pair, 0)
    _compute(0, NCHUNK - 1)

    plsc.subcore_barrier()
    pltpu.sync_copy(deg_sh.at[pl.ds(sid * STRIPE, STRIPE)],
                    deg_hbm.at[cid, pl.ds(sid * STRIPE, STRIPE)])


@functools.partial(
    pl.kernel,
    out_type=jax.ShapeDtypeStruct((NC, N_PAD, D), jnp.float32),
    mesh=_mesh,
    scratch_types=[
        pltpu.VMEM((N_PAD,), jnp.float32),
        pltpu.VMEM((2, 2, C), jnp.int32),
        pltpu.VMEM((2, C), jnp.float32),
        pltpu.VMEM((C,), jnp.float32),
        pltpu.VMEM((C, D), jnp.float32),
        pltpu.VMEM((C, D), jnp.float32),
        pltpu.VMEM_SHARED((N_PAD, D), jnp.float32),
        pltpu.SemaphoreType.DMA,
        pltpu.SemaphoreType.DMA,
    ],
    compiler_params=_sc_params,
)
def _aggregate_kernel(feat_hbm, sd_hbm, ew_in_hbm, dinv_hbm,
                      zeros_hbm, out_hbm,
                      dinv_t, sd, ewc2, normc,
                      ra0, ra1, acc_sh,
                      gsem0, gsem1):
    cid = lax.axis_index("c")
    sid = lax.axis_index("s")
    wid = sid * NC + cid
    ra = (ra0, ra1)
    gsem = (gsem0, gsem1)

    pltpu.sync_copy(dinv_hbm, dinv_t)
    row0 = sid * STRIPE
    pltpu.sync_copy(zeros_hbm, acc_sh.at[pl.ds(row0, STRIPE)])
    plsc.subcore_barrier()

    base = wid * EPT
    tbase = wid * NCHUNK

    def _load_idx(slot, c):
        eb = base + c * C
        pltpu.sync_copy(sd_hbm.at[tbase + c], sd.at[slot])
        pltpu.sync_copy(ew_in_hbm.at[pl.ds(eb, C)], ewc2.at[slot])

    def _issue_gather(slot):
        return pltpu.async_copy(feat_hbm.at[sd.at[slot, 0]], ra[slot],
                                gsem[slot])

    def _compute(slot):
        rab = ra[slot]
        for g in range(C // 16):
            sl = pl.ds(g * 16, 16)
            dis = plsc.load_gather(dinv_t, [sd[slot, 0, sl]])
            normc[sl] = dis * ewc2[slot, sl]

        def _scale(e, carry):
            s16 = plsc.load_gather(normc, [jnp.full((16,), e, jnp.int32)])
            for k in range(D // 16):
                sl2 = pl.ds(k * 16, 16)
                rab[e, sl2] = rab[e, sl2] * s16
            return carry

        lax.fori_loop(0, C, _scale, 0, unroll=4)
        pltpu.sync_copy(rab, acc_sh.at[sd.at[slot, 1]], add=True)

    _load_idx(0, 0)
    _issue_gather(0).wait()

    def _pair(i, carry):
        c0 = i * 2
        _load_idx(1, c0 + 1)
        ha = _issue_gather(1)
        _compute(0)
        ha.wait()
        _load_idx(0, c0 + 2)
        hb = _issue_gather(0)
        _compute(1)
        hb.wait()
        return carry

    lax.fori_loop(0, (NCHUNK - 1) // 2, _pair, 0)
    _compute(0)

    plsc.subcore_barrier()
    for k5 in range(STRIPE // 64):
        sl = pl.ds(row0 + k5 * 64, 64)
        pltpu.sync_copy(acc_sh.at[sl], out_hbm.at[cid, sl])


def _rownorm_body(x_ref, nrm_ref):
    x = x_ref[...]
    nrm_ref[...] = jnp.sqrt(jnp.sum(x * x, axis=1, keepdims=True))


def _row_norms(x_pad):
    rb = 1024
    return pl.pallas_call(
        _rownorm_body,
        grid=(N_PAD // rb,),
        in_specs=[pl.BlockSpec((rb, D), lambda i: (i, 0))],
        out_specs=pl.BlockSpec((rb, 1), lambda i: (i, 0)),
        out_shape=jax.ShapeDtypeStruct((N_PAD, 1), jnp.float32),
    )(x_pad)


def _dinv_body(d0_ref, d1_ref, dinv_ref, dinv2_ref):
    deg = d0_ref[...] + d1_ref[...] + 1.0
    di = jnp.where(deg > 0, lax.rsqrt(deg), 0.0)
    dinv_ref[...] = di
    dinv2_ref[...] = di * di


def _compute_dinv(deg_parts):
    d0 = deg_parts[0].reshape(80, 128)
    d1 = deg_parts[1].reshape(80, 128)
    dinv, dinv2 = pl.pallas_call(
        _dinv_body,
        in_specs=[pl.BlockSpec((80, 128), lambda: (0, 0))] * 2,
        out_specs=[pl.BlockSpec((80, 128), lambda: (0, 0))] * 2,
        out_shape=(jax.ShapeDtypeStruct((80, 128), jnp.float32),) * 2,
    )(d0, d1)
    return dinv.reshape(N_PAD), dinv.reshape(N_PAD, 1), dinv2.reshape(N_PAD, 1)


def _combine_body(p0_ref, p1_ref, f_ref, d1_ref, d2_ref, w_ref, b_ref,
                  out_ref):
    z = d1_ref[...] * (p0_ref[...] + p1_ref[...]) + d2_ref[...] * f_ref[...]
    o = jnp.dot(z, w_ref[...], preferred_element_type=jnp.float32) + b_ref[...]
    out_ref[...] = jnp.maximum(o, 0.0)


def _combine_matmul_relu(parts, feat_pad, dinv1, dinv2, W, b):
    rb = 1024
    return pl.pallas_call(
        _combine_body,
        grid=(N_PAD // rb,),
        in_specs=[
            pl.BlockSpec((rb, D), lambda i: (i, 0)),
            pl.BlockSpec((rb, D), lambda i: (i, 0)),
            pl.BlockSpec((rb, D), lambda i: (i, 0)),
            pl.BlockSpec((rb, 1), lambda i: (i, 0)),
            pl.BlockSpec((rb, 1), lambda i: (i, 0)),
            pl.BlockSpec((D, D), lambda i: (0, 0)),
            pl.BlockSpec((1, D), lambda i: (0, 0)),
        ],
        out_specs=pl.BlockSpec((rb, D), lambda i: (i, 0)),
        out_shape=jax.ShapeDtypeStruct((N_PAD, D), jnp.float32),
    )(parts[0], parts[1], feat_pad, dinv1, dinv2, W, b.reshape(1, D))


def _final_body(p0_ref, p1_ref, f_ref, d1_ref, d2_ref, w_ref, b_ref,
                wc_ref, bc_ref, h_ref, y_ref):
    z = d1_ref[...] * (p0_ref[...] + p1_ref[...]) + d2_ref[...] * f_ref[...]
    o = jnp.dot(z, w_ref[...], preferred_element_type=jnp.float32) + b_ref[...]
    h = jnp.maximum(o, 0.0)
    h_ref[...] = h
    y_ref[...] = jnp.dot(h, wc_ref[...], preferred_element_type=jnp.float32) \
        + bc_ref[...]


def _final_layer(parts, feat_pad, dinv1, dinv2, W, b, Wc, bc):
    rb = 1024
    return pl.pallas_call(
        _final_body,
        grid=(N_PAD // rb,),
        in_specs=[
            pl.BlockSpec((rb, D), lambda i: (i, 0)),
            pl.BlockSpec((rb, D), lambda i: (i, 0)),
            pl.BlockSpec((rb, D), lambda i: (i, 0)),
            pl.BlockSpec((rb, 1), lambda i: (i, 0)),
            pl.BlockSpec((rb, 1), lambda i: (i, 0)),
            pl.BlockSpec((D, D), lambda i: (0, 0)),
            pl.BlockSpec((1, D), lambda i: (0, 0)),
            pl.BlockSpec((D, 1), lambda i: (0, 0)),
            pl.BlockSpec((1, 1), lambda i: (0, 0)),
        ],
        out_specs=[
            pl.BlockSpec((rb, D), lambda i: (i, 0)),
            pl.BlockSpec((rb, 1), lambda i: (i, 0)),
        ],
        out_shape=(
            jax.ShapeDtypeStruct((N_PAD, D), jnp.float32),
            jax.ShapeDtypeStruct((N_PAD, 1), jnp.float32),
        ),
    )(parts[0], parts[1], feat_pad, dinv1, dinv2, W, b.reshape(1, D),
      Wc, bc.reshape(1, 1))


def kernel(x, edge_index, sensitive_attr, W1, b1, W2, b2, Wc, bc):
    edge_index = edge_index.astype(jnp.int32)
    sd = jnp.transpose(edge_index.reshape(2, NW, NCHUNK, C), (1, 2, 0, 3))
    sd = sd.reshape(NW * NCHUNK, 2, C)
    sa = sensitive_attr.astype(jnp.int32)
    sa_pad = jnp.pad(sa, (0, N_PAD - N))
    x_pad = jnp.pad(x, ((0, N_PAD - N), (0, 0)))

    zeros_acc = jnp.zeros((STRIPE, D), jnp.float32)
    nrm = _row_norms(x_pad).reshape(N_PAD)
    ew, deg_parts = _edge_weight_kernel(x_pad, sd, nrm, sa_pad)
    dinv, dinv1, dinv2 = _compute_dinv(deg_parts)

    agg1 = _aggregate_kernel(x_pad, sd, ew, dinv, zeros_acc)
    h1 = _combine_matmul_relu(agg1, x_pad, dinv1, dinv2, W1, b1)

    agg2 = _aggregate_kernel(h1, sd, ew, dinv, zeros_acc)
    h2, y = _final_layer(agg2, h1, dinv1, dinv2, W2, b2, Wc, bc)

    return (y[:N], h2[:N])
